# baseline (speedup 1.0000x reference)
import jax
import jax.numpy as jnp
from jax import lax
from jax.experimental import pallas as pl
from jax.experimental.pallas import tpu as pltpu

EMBED_DIM = 2048
NUM_EXPERTS = 16
TOP_K = 2

BLOCK_T = 2048
BLOCK_K = 512
NK = EMBED_DIM // BLOCK_K


def _router_block(x_ref, w_ref, idx_ref, wgt_ref, logits_ref, acc_ref):
    k = pl.program_id(1)
    partial = jax.lax.dot_general(
        x_ref[...], w_ref[...],
        dimension_numbers=(((1,), (1,)), ((), ())),
        preferred_element_type=jnp.float32,
    )

    @pl.when(k == 0)
    def _init():
        acc_ref[...] = partial

    @pl.when(k != 0)
    def _accum():
        acc_ref[...] += partial

    @pl.when(k == NK - 1)
    def _epilogue():
        logits = acc_ref[...]
        logits_ref[...] = logits
        iota = lax.broadcasted_iota(jnp.int32, logits.shape, 1)
        m1 = jnp.max(logits, axis=-1, keepdims=True)
        i1 = jnp.min(jnp.where(logits == m1, iota, NUM_EXPERTS), axis=-1,
                     keepdims=True)
        masked = jnp.where(iota == i1, -jnp.inf, logits)
        m2 = jnp.max(masked, axis=-1, keepdims=True)
        i2 = jnp.min(jnp.where(masked == m2, iota, NUM_EXPERTS), axis=-1,
                     keepdims=True)
        w1 = jax.nn.sigmoid(m1 - m2)
        idx_ref[...] = jnp.concatenate([i1, i2], axis=-1)
        wgt_ref[...] = jnp.concatenate([w1, 1.0 - w1], axis=-1)


def kernel(x, W):
    n_tokens = x.shape[0]
    grid = (n_tokens // BLOCK_T, NK)
    out_types = (
        jax.ShapeDtypeStruct((n_tokens, TOP_K), jnp.int32),
        jax.ShapeDtypeStruct((n_tokens, TOP_K), jnp.float32),
        jax.ShapeDtypeStruct((n_tokens, NUM_EXPERTS), jnp.float32),
    )
    idx, wgt, logits = pl.pallas_call(
        _router_block,
        grid=grid,
        in_specs=[
            pl.BlockSpec((BLOCK_T, BLOCK_K), lambda i, k: (i, k)),
            pl.BlockSpec((NUM_EXPERTS, BLOCK_K), lambda i, k: (0, k)),
        ],
        out_specs=(
            pl.BlockSpec((BLOCK_T, TOP_K), lambda i, k: (i, 0)),
            pl.BlockSpec((BLOCK_T, TOP_K), lambda i, k: (i, 0)),
            pl.BlockSpec((BLOCK_T, NUM_EXPERTS), lambda i, k: (i, 0)),
        ),
        out_shape=out_types,
        scratch_shapes=[pltpu.VMEM((BLOCK_T, NUM_EXPERTS), jnp.float32)],
    )(x, W)
    return (idx, wgt, logits)


# speedup vs baseline: 1.2959x; 1.2959x over previous
import jax
import jax.numpy as jnp
from jax import lax
from jax.experimental import pallas as pl
from jax.experimental.pallas import tpu as pltpu

EMBED_DIM = 2048
NUM_EXPERTS = 16
TOP_K = 2

BLOCK_T = 2048


def _router_block(x_ref, w_ref, idx_ref, wgt_ref, logits_ref):
    logits = jax.lax.dot_general(
        x_ref[...], w_ref[...],
        dimension_numbers=(((1,), (1,)), ((), ())),
        preferred_element_type=jnp.float32,
    )
    logits_ref[...] = logits

    iota = lax.broadcasted_iota(jnp.int32, logits.shape, 1)
    m1 = jnp.max(logits, axis=-1, keepdims=True)
    i1 = jnp.min(jnp.where(logits == m1, iota, NUM_EXPERTS), axis=-1,
                 keepdims=True)
    masked = jnp.where(iota == i1, -jnp.inf, logits)
    m2 = jnp.max(masked, axis=-1, keepdims=True)
    i2 = jnp.min(jnp.where(masked == m2, iota, NUM_EXPERTS), axis=-1,
                 keepdims=True)
    w1 = jax.nn.sigmoid(m1 - m2)
    idx_ref[...] = jnp.concatenate([i1, i2], axis=-1)
    wgt_ref[...] = jnp.concatenate([w1, 1.0 - w1], axis=-1)


def kernel(x, W):
    n_tokens = x.shape[0]
    grid = (n_tokens // BLOCK_T,)
    out_types = (
        jax.ShapeDtypeStruct((n_tokens, TOP_K), jnp.int32),
        jax.ShapeDtypeStruct((n_tokens, TOP_K), jnp.float32),
        jax.ShapeDtypeStruct((n_tokens, NUM_EXPERTS), jnp.float32),
    )
    idx, wgt, logits = pl.pallas_call(
        _router_block,
        grid=grid,
        in_specs=[
            pl.BlockSpec((BLOCK_T, EMBED_DIM), lambda i: (i, 0)),
            pl.BlockSpec((NUM_EXPERTS, EMBED_DIM), lambda i: (0, 0)),
        ],
        out_specs=(
            pl.BlockSpec((BLOCK_T, TOP_K), lambda i: (i, 0)),
            pl.BlockSpec((BLOCK_T, TOP_K), lambda i: (i, 0)),
            pl.BlockSpec((BLOCK_T, NUM_EXPERTS), lambda i: (i, 0)),
        ),
        out_shape=out_types,
        compiler_params=pltpu.CompilerParams(
            dimension_semantics=("parallel",),
        ),
    )(x, W)
    return (idx, wgt, logits)
